# Initial kernel scaffold; baseline (speedup 1.0000x reference)
import jax
import jax.numpy as jnp
from jax.experimental import pallas as pl


def kernel(x):
    raise NotImplementedError("write your pallas kernel here")


# speedup vs baseline: 2.2752x; 2.2752x over previous
import numpy as np
import jax
import jax.numpy as jnp
from jax.experimental import pallas as pl
from jax.experimental.pallas import tpu as pltpu

_H = 512
_W = 512
_NPIX = float(_H * _W)
_NUM_BINS = 8
_WIDTH = 255.0 / 8.0

_FR = float(-np.sin(2.0 * np.pi * 5 / 8))
_FC = float(np.cos(2.0 * np.pi * 5 / 8) + 1.0)
_A = np.float32(_FR * _FC)
_B = np.float32(_FR * _FR)
_T = np.float32(1.0 - (1.0 - _FR) * _FC)

_MOB = np.zeros((8, _NUM_BINS), dtype=np.float32)
_MOB[0] = [-1.0, 1.0, 0.0, 0.0, 0.0, 0.0, 0.0, 0.0]
_MOB[1] = [-1.0, 0.0, 1.0, 0.0, 0.0, 0.0, 0.0, 0.0]
_MOB[2] = [-1.0, 0.0, 0.0, 0.0, 1.0, 0.0, 0.0, 0.0]
_MOB[3] = [1.0, -1.0, -1.0, 1.0, 0.0, 0.0, 0.0, 0.0]
_MOB[4] = [1.0, -1.0, 0.0, 0.0, -1.0, 1.0, 0.0, 0.0]
_MOB[5] = [1.0, 0.0, -1.0, 0.0, -1.0, 0.0, 1.0, 0.0]
_MOB[6] = [-1.0, 1.0, 1.0, -1.0, 1.0, -1.0, -1.0, 1.0]
_MOB[7] = [1.0, 0.0, 0.0, 0.0, 0.0, 0.0, 0.0, 0.0]


def _lbp_hist_kernel(x_ref, mob_ref, out_ref):
    x = x_ref[0]
    zrow = jnp.zeros((1, _W), jnp.float32)
    zcol = jnp.zeros((_H, 1), jnp.float32)
    rn = jnp.concatenate([x[1:, :], zrow], axis=0)
    xm = jnp.concatenate([zcol, x[:, :-1]], axis=1)
    xp = jnp.concatenate([x[:, 1:], zcol], axis=1)
    rm = jnp.concatenate([zcol, rn[:, :-1]], axis=1)
    rp = jnp.concatenate([rn[:, 1:], zcol], axis=1)

    t = _T * x
    u = _A * rn
    b5 = ((_A * xm + _B * rm + u) >= t).astype(jnp.float32)
    b6 = (rn >= x).astype(jnp.float32)
    b7 = ((_A * xp + _B * rp + u) >= t).astype(jnp.float32)

    p56 = b5 * b6
    p57 = b5 * b7
    p67 = b6 * b7
    p567 = p56 * b7

    mob = mob_ref[...]
    counts = (
        jnp.sum(b5) * mob[0]
        + jnp.sum(b6) * mob[1]
        + jnp.sum(b7) * mob[2]
        + jnp.sum(p56) * mob[3]
        + jnp.sum(p57) * mob[4]
        + jnp.sum(p67) * mob[5]
        + jnp.sum(p567) * mob[6]
        + _NPIX * mob[7]
    )
    dens = (counts / np.float32(_WIDTH)) / np.float32(_NPIX)
    out_ref[0, 0] = dens


def kernel(x):
    B, C, H, W = x.shape
    planes = x.reshape(B * C, H, W)
    out = pl.pallas_call(
        _lbp_hist_kernel,
        grid=(B * C,),
        in_specs=[
            pl.BlockSpec((1, H, W), lambda i: (i, 0, 0)),
            pl.BlockSpec((8, _NUM_BINS), lambda i: (0, 0)),
        ],
        out_specs=pl.BlockSpec((1, 1, _NUM_BINS), lambda i: (i, 0, 0)),
        out_shape=jax.ShapeDtypeStruct((B * C, 1, _NUM_BINS), jnp.float32),
    )(planes, jnp.asarray(_MOB))
    return out.reshape(B, C * _NUM_BINS)


# speedup vs baseline: 3.3023x; 1.4514x over previous
import numpy as np
import jax
import jax.numpy as jnp
from jax.experimental import pallas as pl
from jax.experimental.pallas import tpu as pltpu

_H = 512
_W = 512
_NPIX = float(_H * _W)
_NUM_BINS = 8
_WIDTH = 255.0 / 8.0

_FR = float(-np.sin(2.0 * np.pi * 5 / 8))
_FC = float(np.cos(2.0 * np.pi * 5 / 8) + 1.0)
_A = np.float32(_FR * _FC)
_B = np.float32(_FR * _FR)
_T = np.float32(1.0 - (1.0 - _FR) * _FC)

_MOB = np.zeros((8, _NUM_BINS), dtype=np.float32)
_MOB[0] = [-1.0, 1.0, 0.0, 0.0, 0.0, 0.0, 0.0, 0.0]
_MOB[1] = [-1.0, 0.0, 1.0, 0.0, 0.0, 0.0, 0.0, 0.0]
_MOB[2] = [-1.0, 0.0, 0.0, 0.0, 1.0, 0.0, 0.0, 0.0]
_MOB[3] = [1.0, -1.0, -1.0, 1.0, 0.0, 0.0, 0.0, 0.0]
_MOB[4] = [1.0, -1.0, 0.0, 0.0, -1.0, 1.0, 0.0, 0.0]
_MOB[5] = [1.0, 0.0, -1.0, 0.0, -1.0, 0.0, 1.0, 0.0]
_MOB[6] = [-1.0, 1.0, 1.0, -1.0, 1.0, -1.0, -1.0, 1.0]
_MOB[7] = [1.0, 0.0, 0.0, 0.0, 0.0, 0.0, 0.0, 0.0]


def _lbp_hist_kernel(x_ref, mob_ref, out_ref):
    x = x_ref[0]
    zrow = jnp.zeros((1, _W), jnp.float32)
    zcol = jnp.zeros((_H, 1), jnp.float32)
    rn = jnp.concatenate([x[1:, :], zrow], axis=0)
    g = _A * x + _B * rn
    gm = jnp.concatenate([zcol, g[:, :-1]], axis=1)
    gp = jnp.concatenate([g[:, 1:], zcol], axis=1)
    w = _T * x - _A * rn

    b5 = (gm >= w).astype(jnp.float32)
    b6 = (rn >= x).astype(jnp.float32)
    b7 = (gp >= w).astype(jnp.float32)

    p56 = b5 * b6
    p57 = b5 * b7
    p67 = b6 * b7
    p567 = p56 * b7

    mob = mob_ref[...]
    counts = (
        jnp.sum(b5) * mob[0]
        + jnp.sum(b6) * mob[1]
        + jnp.sum(b7) * mob[2]
        + jnp.sum(p56) * mob[3]
        + jnp.sum(p57) * mob[4]
        + jnp.sum(p67) * mob[5]
        + jnp.sum(p567) * mob[6]
        + _NPIX * mob[7]
    )
    dens = (counts / np.float32(_WIDTH)) / np.float32(_NPIX)
    out_ref[0, 0] = dens


def kernel(x):
    B, C, H, W = x.shape
    planes = x.reshape(B * C, H, W)
    out = pl.pallas_call(
        _lbp_hist_kernel,
        grid=(B * C,),
        in_specs=[
            pl.BlockSpec((1, H, W), lambda i: (i, 0, 0)),
            pl.BlockSpec((8, _NUM_BINS), lambda i: (0, 0)),
        ],
        out_specs=pl.BlockSpec((1, 1, _NUM_BINS), lambda i: (i, 0, 0)),
        out_shape=jax.ShapeDtypeStruct((B * C, 1, _NUM_BINS), jnp.float32),
    )(planes, jnp.asarray(_MOB))
    return out.reshape(B, C * _NUM_BINS)


# speedup vs baseline: 3.3097x; 1.0022x over previous
import numpy as np
import jax
import jax.numpy as jnp
from jax.experimental import pallas as pl
from jax.experimental.pallas import tpu as pltpu

_H = 512
_W = 512
_NPIX = float(_H * _W)
_NUM_BINS = 8
_WIDTH = 255.0 / 8.0

_FR = float(-np.sin(2.0 * np.pi * 5 / 8))
_FC = float(np.cos(2.0 * np.pi * 5 / 8) + 1.0)
_A = np.float32(_FR * _FC)
_B = np.float32(_FR * _FR)
_T = np.float32(1.0 - (1.0 - _FR) * _FC)

_MOB = np.zeros((8, _NUM_BINS), dtype=np.float32)
_MOB[0] = [-1.0, 1.0, 0.0, 0.0, 0.0, 0.0, 0.0, 0.0]
_MOB[1] = [-1.0, 0.0, 1.0, 0.0, 0.0, 0.0, 0.0, 0.0]
_MOB[2] = [-1.0, 0.0, 0.0, 0.0, 1.0, 0.0, 0.0, 0.0]
_MOB[3] = [1.0, -1.0, -1.0, 1.0, 0.0, 0.0, 0.0, 0.0]
_MOB[4] = [1.0, -1.0, 0.0, 0.0, -1.0, 1.0, 0.0, 0.0]
_MOB[5] = [1.0, 0.0, -1.0, 0.0, -1.0, 0.0, 1.0, 0.0]
_MOB[6] = [-1.0, 1.0, 1.0, -1.0, 1.0, -1.0, -1.0, 1.0]
_MOB[7] = [1.0, 0.0, 0.0, 0.0, 0.0, 0.0, 0.0, 0.0]


def _lbp_hist_kernel(x_ref, mob_ref, out_ref):
    x = x_ref[0]
    zrow = jnp.zeros((1, _W), jnp.float32)
    zcol = jnp.zeros((_H, 1), jnp.float32)
    rn = jnp.concatenate([x[1:, :], zrow], axis=0)
    g = _A * x + _B * rn
    gm = jnp.concatenate([zcol, g[:, :-1]], axis=1)
    gp = jnp.concatenate([g[:, 1:], zcol], axis=1)
    w = _T * x - _A * rn

    b5 = (gm >= w).astype(jnp.float32)
    b6 = (rn >= x).astype(jnp.float32)
    b7 = (gp >= w).astype(jnp.float32)

    p56 = b5 * b6
    p57 = b5 * b7
    p67 = b6 * b7
    p567 = p56 * b7

    mob = mob_ref[...]
    counts = (
        jnp.sum(b5) * mob[0]
        + jnp.sum(b6) * mob[1]
        + jnp.sum(b7) * mob[2]
        + jnp.sum(p56) * mob[3]
        + jnp.sum(p57) * mob[4]
        + jnp.sum(p67) * mob[5]
        + jnp.sum(p567) * mob[6]
        + _NPIX * mob[7]
    )
    dens = (counts / np.float32(_WIDTH)) / np.float32(_NPIX)
    out_ref[0, 0] = dens


def kernel(x):
    B, C, H, W = x.shape
    planes = x.reshape(B * C, H, W)
    out = pl.pallas_call(
        _lbp_hist_kernel,
        grid=(B * C,),
        in_specs=[
            pl.BlockSpec((1, H, W), lambda i: (i, 0, 0)),
            pl.BlockSpec((8, _NUM_BINS), lambda i: (0, 0)),
        ],
        out_specs=pl.BlockSpec((1, 1, _NUM_BINS), lambda i: (i, 0, 0)),
        out_shape=jax.ShapeDtypeStruct((B * C, 1, _NUM_BINS), jnp.float32),
        compiler_params=pltpu.CompilerParams(
            dimension_semantics=("parallel",),
        ),
    )(planes, jnp.asarray(_MOB))
    return out.reshape(B, C * _NUM_BINS)
